# Initial kernel scaffold; baseline (speedup 1.0000x reference)
import functools

import jax
import jax.numpy as jnp
from jax import lax
from jax.experimental import pallas as pl
from jax.experimental.pallas import tpu as pltpu
from jax.experimental.pallas import tpu_sc as plsc

N = 10000
HID = 256
HEADS = 8
PH = 32
OUT_CH = 64

NC = 2
NS = 16
L = 16
NW = NC * NS
B = 128
N_ACC = 12000
DUMMY = N
WST = 16
f32 = jnp.float32
i32 = jnp.int32

ROWS_PT = N_ACC // NS


def _ceil_to(a, m):
  return ((a + m - 1) // m) * m


def _prep_edges(ei, ep):
  loop = jnp.arange(N, dtype=i32)
  src = jnp.concatenate([ei[0].astype(i32), loop])
  dst = jnp.concatenate([ei[1].astype(i32), loop])
  pad = ep - src.shape[0]
  src = jnp.concatenate([src, jnp.zeros((pad,), i32)])
  dst = jnp.concatenate([dst, jnp.full((pad,), DUMMY, i32)])
  return src, dst


_DN = (((1,), (1,)), ((), ()))


def _proj0_body(x_ref, wl_ref, wr_ref, xl_ref, xr_ref):
  xb = x_ref[...]
  xl_ref[...] = lax.dot_general(xb, wl_ref[...], _DN,
                                preferred_element_type=f32)
  xr_ref[...] = lax.dot_general(xb, wr_ref[...], _DN,
                                preferred_element_type=f32)


def _tc_project0(x, wl, wr):
  grid = (5, 2)
  return pl.pallas_call(
      _proj0_body,
      grid=grid,
      in_specs=[
          pl.BlockSpec((2000, 128), lambda i, j: (i, 0)),
          pl.BlockSpec((128, 128), lambda i, j: (j, 0)),
          pl.BlockSpec((128, 128), lambda i, j: (j, 0)),
      ],
      out_specs=[
          pl.BlockSpec((2000, 128), lambda i, j: (j * 5 + i, 0)),
          pl.BlockSpec((2000, 128), lambda i, j: (j * 5 + i, 0)),
      ],
      out_shape=[
          jax.ShapeDtypeStruct((2 * N, 128), f32),
          jax.ShapeDtypeStruct((2 * N, 128), f32),
      ],
  )(x, wl, wr)


def _mid_body(lo_ref, hi_ref, b0_ref, wl_ref, wr_ref, xl_ref, xr_ref):
  h = jnp.concatenate([lo_ref[...], hi_ref[...]], axis=1) + b0_ref[...]
  h = jnp.where(h > 0, h, jnp.exp(h) - 1.0)
  xl_ref[...] = lax.dot_general(h, wl_ref[...], _DN,
                                preferred_element_type=f32)
  xr_ref[...] = lax.dot_general(h, wr_ref[...], _DN,
                                preferred_element_type=f32)


def _tc_mid(h0, b0, wl1, wr1):
  grid = (5,)
  return pl.pallas_call(
      _mid_body,
      grid=grid,
      in_specs=[
          pl.BlockSpec((2000, 128), lambda i: (i, 0)),
          pl.BlockSpec((2000, 128), lambda i: (i + N_ACC // 2000, 0)),
          pl.BlockSpec((1, HID), lambda i: (0, 0)),
          pl.BlockSpec((OUT_CH, HID), lambda i: (0, 0)),
          pl.BlockSpec((OUT_CH, HID), lambda i: (0, 0)),
      ],
      out_specs=[
          pl.BlockSpec((2000, OUT_CH), lambda i: (i, 0)),
          pl.BlockSpec((2000, OUT_CH), lambda i: (i, 0)),
      ],
      out_shape=[
          jax.ShapeDtypeStruct((N, OUT_CH), f32),
          jax.ShapeDtypeStruct((N, OUT_CH), f32),
      ],
  )(h0, h0, b0.reshape(1, HID), wl1, wr1)


def _final_body(p0_ref, p1_ref, b1_ref, o_ref):
  z = p0_ref[...] + p1_ref[...] + b1_ref[...]
  m = jnp.max(z, axis=1, keepdims=True)
  s = jnp.log(jnp.sum(jnp.exp(z - m), axis=1, keepdims=True))
  o_ref[...] = z - m - s


def _tc_final(out1, b1):
  grid = (5,)
  return pl.pallas_call(
      _final_body,
      grid=grid,
      in_specs=[
          pl.BlockSpec((2000, OUT_CH), lambda i: (i, 0)),
          pl.BlockSpec((2000, OUT_CH), lambda i: (i + N_ACC // 2000, 0)),
          pl.BlockSpec((1, OUT_CH), lambda i: (0, 0)),
      ],
      out_specs=pl.BlockSpec((2000, OUT_CH), lambda i: (i, 0)),
      out_shape=jax.ShapeDtypeStruct((N, OUT_CH), f32),
  )(out1, out1, b1.reshape(1, OUT_CH))


_MESH = dict(core_axis_name="c", subcore_axis_name="s", num_cores=NC,
             num_subcores=NS)


def _zero_rows(z_h, sh, r0):
  pltpu.sync_copy(z_h.at[pl.ds(r0, ROWS_PT)], sh.at[pl.ds(r0, ROWS_PT)])


@functools.lru_cache(maxsize=None)
def _make_pass_a(ep, heads, ph, tw, ntab):
  ca = ep // NW
  nblk = ca // B

  def body(src_h, dst_h, xl_h, xr_h, att_h, z_h, ex_h, den_h, *scr):
    sidx, didx = scr[0], scr[1]
    xlb = scr[2:2 + ntab]
    xrb = scr[2 + ntab:2 + 2 * ntab]
    if ntab == 2:
      sidx2, didx2 = scr[6], scr[7]
      exb, attv, den_sh, sem = scr[8], scr[9], scr[10], scr[11]
    else:
      exb, attv, den_sh, sem = scr[2 + 2 * ntab:2 + 2 * ntab + 4]
    cid = lax.axis_index("c")
    sid = lax.axis_index("s")
    wid = sid * NC + cid
    r0 = sid * ROWS_PT
    pltpu.sync_copy(att_h, attv)
    _zero_rows(z_h, den_sh, r0)
    zero16 = jnp.zeros((L,), f32)
    for r in range(B):
      exb[r, pl.ds(0, WST)] = zero16
    plsc.subcore_barrier()

    def blk(i, carry):
      base = wid * ca + i * B
      pltpu.sync_copy(src_h.at[pl.ds(base, B)], sidx)
      pltpu.sync_copy(dst_h.at[pl.ds(base, B)], didx)
      if ntab == 2:
        for k in range(B // L):
          sidx2[pl.ds(k * L, L)] = sidx[pl.ds(k * L, L)] + N
          didx2[pl.ds(k * L, L)] = didx[pl.ds(k * L, L)] + N
      cps = [pltpu.async_copy(xl_h.at[sidx], xlb[0], sem),
             pltpu.async_copy(xr_h.at[didx], xrb[0], sem)]
      if ntab == 2:
        cps.append(pltpu.async_copy(xl_h.at[sidx2], xlb[1], sem))
        cps.append(pltpu.async_copy(xr_h.at[didx2], xrb[1], sem))
      for cp in cps:
        cp.wait()

      def grp(g, c2):
        eidx = lax.iota(i32, L) + g * L
        for h in range(heads):
          acc = jnp.zeros((L,), f32)
          for c in range(ph):
            f = h * ph + c
            t, fl = divmod(f, tw)
            flv = jnp.full((L,), fl, i32)
            s = plsc.load_gather(xlb[t], [eidx, flv]) + \
                plsc.load_gather(xrb[t], [eidx, flv])
            s = jnp.maximum(s, s * 0.2)
            acc = acc + s * attv[f]
          plsc.store_scatter(exb, [eidx, jnp.full((L,), h, i32)],
                             jnp.exp(acc))
        return c2
      lax.fori_loop(0, B // L, grp, 0)
      pltpu.sync_copy(exb, ex_h.at[pl.ds(base, B)])
      pltpu.sync_copy(exb, den_sh.at[didx], add=True)
      return carry
    lax.fori_loop(0, nblk, blk, 0)
    plsc.subcore_barrier()
    pltpu.sync_copy(den_sh.at[pl.ds(r0, ROWS_PT)],
                    den_h.at[pl.ds(cid * N_ACC + r0, ROWS_PT)])

  scratch = [pltpu.VMEM((B,), i32), pltpu.VMEM((B,), i32)]
  scratch += [pltpu.VMEM((B, tw), f32) for _ in range(2 * ntab)]
  if ntab == 2:
    scratch += [pltpu.VMEM((B,), i32), pltpu.VMEM((B,), i32)]
  scratch += [pltpu.VMEM((B, WST), f32), pltpu.VMEM((heads * ph,), f32),
              pltpu.VMEM_SHARED((N_ACC, WST), f32), pltpu.SemaphoreType.DMA]

  return pl.kernel(
      body,
      out_type=[jax.ShapeDtypeStruct((ep, WST), f32),
                jax.ShapeDtypeStruct((2 * N_ACC, WST), f32)],
      mesh=plsc.VectorSubcoreMesh(**_MESH),
      scratch_types=scratch,
  )


@functools.lru_cache(maxsize=None)
def _make_pass_b0(ep):
  cb = ep // NS
  nblk = cb // B
  tw = 128

  def body(src_h, dst_h, xl_h, ex_h, den_h, z_h, out_h,
           sidx, didx, gidx, didxb, xlb, exb, d0b, d1b, msgb, out_sh, sem):
    cid = lax.axis_index("c")
    sid = lax.axis_index("s")
    r0 = sid * ROWS_PT
    _zero_rows(z_h, out_sh, r0)
    plsc.subcore_barrier()
    hbase = cid * (HEADS // NC)

    def blk(i, carry):
      base = sid * cb + i * B
      pltpu.sync_copy(src_h.at[pl.ds(base, B)], sidx)
      pltpu.sync_copy(dst_h.at[pl.ds(base, B)], didx)
      coff = cid * N
      for k in range(B // L):
        gidx[pl.ds(k * L, L)] = sidx[pl.ds(k * L, L)] + coff
        didxb[pl.ds(k * L, L)] = didx[pl.ds(k * L, L)] + N_ACC
      cps = [pltpu.async_copy(xl_h.at[gidx], xlb, sem),
             pltpu.async_copy(den_h.at[didx], d0b, sem),
             pltpu.async_copy(den_h.at[didxb], d1b, sem)]
      pltpu.sync_copy(ex_h.at[pl.ds(base, B)], exb)
      for cp in cps:
        cp.wait()

      def grp(g, c2):
        eidx = lax.iota(i32, L) + g * L
        for hl in range(HEADS // NC):
          hv = jnp.full((L,), hbase + hl, i32)
          ev = plsc.load_gather(exb, [eidx, hv])
          dv = plsc.load_gather(d0b, [eidx, hv]) + \
              plsc.load_gather(d1b, [eidx, hv])
          al = ev / (dv + 1e-16)
          for c in range(PH):
            flv = jnp.full((L,), hl * PH + c, i32)
            v = plsc.load_gather(xlb, [eidx, flv]) * al
            plsc.store_scatter(msgb, [eidx, flv], v)
        return c2
      lax.fori_loop(0, B // L, grp, 0)
      pltpu.sync_copy(msgb, out_sh.at[didx], add=True)
      return carry
    lax.fori_loop(0, nblk, blk, 0)
    plsc.subcore_barrier()
    pltpu.sync_copy(out_sh.at[pl.ds(r0, ROWS_PT)],
                    out_h.at[pl.ds(cid * N_ACC + r0, ROWS_PT)])

  scratch = [pltpu.VMEM((B,), i32), pltpu.VMEM((B,), i32),
             pltpu.VMEM((B,), i32), pltpu.VMEM((B,), i32),
             pltpu.VMEM((B, tw), f32), pltpu.VMEM((B, WST), f32),
             pltpu.VMEM((B, WST), f32), pltpu.VMEM((B, WST), f32),
             pltpu.VMEM((B, tw), f32),
             pltpu.VMEM_SHARED((N_ACC, tw), f32), pltpu.SemaphoreType.DMA]

  return pl.kernel(
      body,
      out_type=jax.ShapeDtypeStruct((2 * N_ACC, tw), f32),
      mesh=plsc.VectorSubcoreMesh(**_MESH),
      scratch_types=scratch,
  )


@functools.lru_cache(maxsize=None)
def _make_pass_b1(ep):
  ca = ep // NW
  nblk = ca // B
  tw = OUT_CH

  def body(src_h, dst_h, xl_h, ex_h, den_h, z_h, out_h,
           sidx, didx, didxb, xlb, exb, d0b, d1b, msgb, out_sh, sem):
    cid = lax.axis_index("c")
    sid = lax.axis_index("s")
    wid = sid * NC + cid
    r0 = sid * ROWS_PT
    _zero_rows(z_h, out_sh, r0)
    plsc.subcore_barrier()

    def blk(i, carry):
      base = wid * ca + i * B
      pltpu.sync_copy(src_h.at[pl.ds(base, B)], sidx)
      pltpu.sync_copy(dst_h.at[pl.ds(base, B)], didx)
      for k in range(B // L):
        didxb[pl.ds(k * L, L)] = didx[pl.ds(k * L, L)] + N_ACC
      cps = [pltpu.async_copy(xl_h.at[sidx], xlb, sem),
             pltpu.async_copy(den_h.at[didx], d0b, sem),
             pltpu.async_copy(den_h.at[didxb], d1b, sem)]
      pltpu.sync_copy(ex_h.at[pl.ds(base, B)], exb)
      for cp in cps:
        cp.wait()

      def grp(g, c2):
        eidx = lax.iota(i32, L) + g * L
        hv = jnp.zeros((L,), i32)
        ev = plsc.load_gather(exb, [eidx, hv])
        dv = plsc.load_gather(d0b, [eidx, hv]) + \
            plsc.load_gather(d1b, [eidx, hv])
        al = ev / (dv + 1e-16)
        for c in range(tw):
          flv = jnp.full((L,), c, i32)
          v = plsc.load_gather(xlb, [eidx, flv]) * al
          plsc.store_scatter(msgb, [eidx, flv], v)
        return c2
      lax.fori_loop(0, B // L, grp, 0)
      pltpu.sync_copy(msgb, out_sh.at[didx], add=True)
      return carry
    lax.fori_loop(0, nblk, blk, 0)
    plsc.subcore_barrier()
    pltpu.sync_copy(out_sh.at[pl.ds(r0, ROWS_PT)],
                    out_h.at[pl.ds(cid * N_ACC + r0, ROWS_PT)])

  scratch = [pltpu.VMEM((B,), i32), pltpu.VMEM((B,), i32),
             pltpu.VMEM((B,), i32),
             pltpu.VMEM((B, tw), f32), pltpu.VMEM((B, WST), f32),
             pltpu.VMEM((B, WST), f32), pltpu.VMEM((B, WST), f32),
             pltpu.VMEM((B, tw), f32),
             pltpu.VMEM_SHARED((N_ACC, tw), f32), pltpu.SemaphoreType.DMA]

  return pl.kernel(
      body,
      out_type=jax.ShapeDtypeStruct((2 * N_ACC, tw), f32),
      mesh=plsc.VectorSubcoreMesh(**_MESH),
      scratch_types=scratch,
  )


def kernel(x, edge_index0, edge_index1, W_l0, W_r0, att0, b0,
           W_l1, W_r1, att1, b1):
  ep0 = _ceil_to(edge_index0.shape[1] + N, NW * B)
  ep1 = _ceil_to(edge_index1.shape[1] + N, NW * B)
  src0, dst0 = _prep_edges(edge_index0, ep0)
  src1, dst1 = _prep_edges(edge_index1, ep1)

  z16 = jnp.zeros((N_ACC, WST), f32)
  z64 = jnp.zeros((N_ACC, OUT_CH), f32)
  z128 = jnp.zeros((N_ACC, 128), f32)

  xl0, xr0 = _tc_project0(x, W_l0, W_r0)
  ex0, den0 = _make_pass_a(ep0, HEADS, PH, 128, 2)(
      src0, dst0, xl0, xr0, att0.reshape(-1), z16)
  h0 = _make_pass_b0(ep0)(src0, dst0, xl0, ex0, den0, z128)

  xl1, xr1 = _tc_mid(h0, b0, W_l1, W_r1)
  ex1, den1 = _make_pass_a(ep1, 1, OUT_CH, OUT_CH, 1)(
      src1, dst1, xl1, xr1, att1.reshape(-1), z16)
  out1 = _make_pass_b1(ep1)(src1, dst1, xl1, ex1, den1, z64)

  return _tc_final(out1, b1)


# speedup vs baseline: 5.7891x; 5.7891x over previous
import functools

import jax
import jax.numpy as jnp
from jax import lax
from jax.experimental import pallas as pl
from jax.experimental.pallas import tpu as pltpu
from jax.experimental.pallas import tpu_sc as plsc

N = 10000
HID = 256
HEADS = 8
PH = 32
OUT_CH = 64

NC = 2
NS = 16
L = 16
NW = NC * NS
B = 128
N_ACC = 10240
DUMMY = N
WST = 16
f32 = jnp.float32
i32 = jnp.int32

ROWS_PT = N_ACC // NS


def _ceil_to(a, m):
  return ((a + m - 1) // m) * m


def _prep_edges(ei, ep):
  loop = jnp.arange(N, dtype=i32)
  src = jnp.concatenate([ei[0].astype(i32), loop])
  dst = jnp.concatenate([ei[1].astype(i32), loop])
  pad = ep - src.shape[0]
  src = jnp.concatenate([src, jnp.zeros((pad,), i32)])
  dst = jnp.concatenate([dst, jnp.full((pad,), DUMMY, i32)])
  return src, dst


_DN = (((1,), (1,)), ((), ()))


def _proj0_body(x_ref, wl_ref, wr_ref, xl_ref, xr_ref):
  xb = x_ref[...]
  xl_ref[...] = lax.dot_general(xb, wl_ref[...], _DN,
                                preferred_element_type=f32)
  xr_ref[...] = lax.dot_general(xb, wr_ref[...], _DN,
                                preferred_element_type=f32)


def _tc_project0(x, wl, wr):
  grid = (5, 2)
  return pl.pallas_call(
      _proj0_body,
      grid=grid,
      in_specs=[
          pl.BlockSpec((2000, 128), lambda i, j: (i, 0)),
          pl.BlockSpec((128, 128), lambda i, j: (j, 0)),
          pl.BlockSpec((128, 128), lambda i, j: (j, 0)),
      ],
      out_specs=[
          pl.BlockSpec((2000, 128), lambda i, j: (j * 5 + i, 0)),
          pl.BlockSpec((2000, 128), lambda i, j: (j * 5 + i, 0)),
      ],
      out_shape=[
          jax.ShapeDtypeStruct((2 * N, 128), f32),
          jax.ShapeDtypeStruct((2 * N, 128), f32),
      ],
  )(x, wl, wr)


def _mid_body(lo_ref, hi_ref, b0_ref, wl_ref, wr_ref, xl_ref, xr_ref):
  h = jnp.concatenate([lo_ref[...], hi_ref[...]], axis=1) + b0_ref[...]
  h = jnp.where(h > 0, h, jnp.exp(h) - 1.0)
  xl_ref[...] = lax.dot_general(h, wl_ref[...], _DN,
                                preferred_element_type=f32)
  xr_ref[...] = lax.dot_general(h, wr_ref[...], _DN,
                                preferred_element_type=f32)


def _tc_mid(h0, b0, wl1, wr1):
  grid = (5,)
  return pl.pallas_call(
      _mid_body,
      grid=grid,
      in_specs=[
          pl.BlockSpec((2048, 128), lambda i: (i, 0)),
          pl.BlockSpec((2048, 128), lambda i: (i + N_ACC // 2048, 0)),
          pl.BlockSpec((1, HID), lambda i: (0, 0)),
          pl.BlockSpec((OUT_CH, HID), lambda i: (0, 0)),
          pl.BlockSpec((OUT_CH, HID), lambda i: (0, 0)),
      ],
      out_specs=[
          pl.BlockSpec((2048, OUT_CH), lambda i: (i, 0)),
          pl.BlockSpec((2048, OUT_CH), lambda i: (i, 0)),
      ],
      out_shape=[
          jax.ShapeDtypeStruct((N, OUT_CH), f32),
          jax.ShapeDtypeStruct((N, OUT_CH), f32),
      ],
  )(h0, h0, b0.reshape(1, HID), wl1, wr1)


def _final_body(p0_ref, p1_ref, b1_ref, o_ref):
  z = p0_ref[...] + p1_ref[...] + b1_ref[...]
  m = jnp.max(z, axis=1, keepdims=True)
  s = jnp.log(jnp.sum(jnp.exp(z - m), axis=1, keepdims=True))
  o_ref[...] = z - m - s


def _tc_final(out1, b1):
  grid = (5,)
  return pl.pallas_call(
      _final_body,
      grid=grid,
      in_specs=[
          pl.BlockSpec((2048, OUT_CH), lambda i: (i, 0)),
          pl.BlockSpec((2048, OUT_CH), lambda i: (i + N_ACC // 2048, 0)),
          pl.BlockSpec((1, OUT_CH), lambda i: (0, 0)),
      ],
      out_specs=pl.BlockSpec((2048, OUT_CH), lambda i: (i, 0)),
      out_shape=jax.ShapeDtypeStruct((N, OUT_CH), f32),
  )(out1, out1, b1.reshape(1, OUT_CH))


_MESH = dict(core_axis_name="c", subcore_axis_name="s", num_cores=NC,
             num_subcores=NS)


def _zero_rows(z_h, sh, r0):
  pltpu.sync_copy(z_h.at[pl.ds(r0, ROWS_PT)], sh.at[pl.ds(r0, ROWS_PT)])


@functools.lru_cache(maxsize=None)
def _make_pass_a(ep, heads, ph, tw, ntab):
  ca = ep // NW
  nblk = ca // B

  def body(src_h, dst_h, xl_h, xr_h, att_h, z_h, ex_h, den_h, *scr):
    sidx, didx = scr[0], scr[1]
    xlb = scr[2:2 + ntab]
    xrb = scr[2 + ntab:2 + 2 * ntab]
    if ntab == 2:
      sidx2, didx2 = scr[6], scr[7]
      exb, attv, den_sh, sem = scr[8], scr[9], scr[10], scr[11]
    else:
      exb, attv, den_sh, sem = scr[2 + 2 * ntab:2 + 2 * ntab + 4]
    cid = lax.axis_index("c")
    sid = lax.axis_index("s")
    wid = sid * NC + cid
    r0 = sid * ROWS_PT
    pltpu.sync_copy(att_h, attv)
    _zero_rows(z_h, den_sh, r0)
    zero16 = jnp.zeros((L,), f32)
    for r in range(B):
      exb[r, pl.ds(0, WST)] = zero16
    plsc.subcore_barrier()

    def blk(i, carry):
      base = wid * ca + i * B
      pltpu.sync_copy(src_h.at[pl.ds(base, B)], sidx)
      pltpu.sync_copy(dst_h.at[pl.ds(base, B)], didx)
      if ntab == 2:
        for k in range(B // L):
          sidx2[pl.ds(k * L, L)] = sidx[pl.ds(k * L, L)] + N
          didx2[pl.ds(k * L, L)] = didx[pl.ds(k * L, L)] + N
      cps = [pltpu.async_copy(xl_h.at[sidx], xlb[0], sem),
             pltpu.async_copy(xr_h.at[didx], xrb[0], sem)]
      if ntab == 2:
        cps.append(pltpu.async_copy(xl_h.at[sidx2], xlb[1], sem))
        cps.append(pltpu.async_copy(xr_h.at[didx2], xrb[1], sem))
      for cp in cps:
        cp.wait()

      def grp(g, c2):
        eidx = lax.iota(i32, L) + g * L
        for h in range(heads):
          acc = jnp.zeros((L,), f32)
          for k in range(ph // L):
            av = attv[pl.ds(h * ph + k * L, L)]
            for j in range(L):
              f = h * ph + k * L + j
              t, fl = divmod(f, tw)
              flv = jnp.full((L,), fl, i32)
              s = plsc.load_gather(xlb[t], [eidx, flv]) + \
                  plsc.load_gather(xrb[t], [eidx, flv])
              s = jnp.maximum(s, s * 0.2)
              acc = acc + s * av[j]
          plsc.store_scatter(exb, [eidx, jnp.full((L,), h, i32)],
                             jnp.exp(acc))
        return c2
      lax.fori_loop(0, B // L, grp, 0)
      pltpu.sync_copy(exb, ex_h.at[pl.ds(base, B)])
      pltpu.sync_copy(exb, den_sh.at[didx], add=True)
      return carry
    lax.fori_loop(0, nblk, blk, 0)
    plsc.subcore_barrier()
    pltpu.sync_copy(den_sh.at[pl.ds(r0, ROWS_PT)],
                    den_h.at[pl.ds(cid * N_ACC + r0, ROWS_PT)])

  scratch = [pltpu.VMEM((B,), i32), pltpu.VMEM((B,), i32)]
  scratch += [pltpu.VMEM((B, tw), f32) for _ in range(2 * ntab)]
  if ntab == 2:
    scratch += [pltpu.VMEM((B,), i32), pltpu.VMEM((B,), i32)]
  scratch += [pltpu.VMEM((B, WST), f32), pltpu.VMEM((heads * ph,), f32),
              pltpu.VMEM_SHARED((N_ACC, WST), f32), pltpu.SemaphoreType.DMA]

  return pl.kernel(
      body,
      out_type=[jax.ShapeDtypeStruct((ep, WST), f32),
                jax.ShapeDtypeStruct((2 * N_ACC, WST), f32)],
      mesh=plsc.VectorSubcoreMesh(**_MESH),
      scratch_types=scratch,
      compiler_params=pltpu.CompilerParams(needs_layout_passes=False, use_tc_tiling_on_sc=False),
  )


@functools.lru_cache(maxsize=None)
def _make_pass_b0(ep):
  cb = ep // NS
  nblk = cb // B
  tw = 128

  def body(src_h, dst_h, xl_h, ex_h, den_h, z_h, out_h,
           sidx, didx, gidx, didxb, xlb, exb, d0b, d1b, msgb, out_sh, sem):
    cid = lax.axis_index("c")
    sid = lax.axis_index("s")
    r0 = sid * ROWS_PT
    _zero_rows(z_h, out_sh, r0)
    plsc.subcore_barrier()
    hbase = cid * (HEADS // NC)

    def blk(i, carry):
      base = sid * cb + i * B
      pltpu.sync_copy(src_h.at[pl.ds(base, B)], sidx)
      pltpu.sync_copy(dst_h.at[pl.ds(base, B)], didx)
      coff = cid * N
      for k in range(B // L):
        gidx[pl.ds(k * L, L)] = sidx[pl.ds(k * L, L)] + coff
        didxb[pl.ds(k * L, L)] = didx[pl.ds(k * L, L)] + N_ACC
      cps = [pltpu.async_copy(xl_h.at[gidx], xlb, sem),
             pltpu.async_copy(den_h.at[didx], d0b, sem),
             pltpu.async_copy(den_h.at[didxb], d1b, sem)]
      pltpu.sync_copy(ex_h.at[pl.ds(base, B)], exb)
      for cp in cps:
        cp.wait()

      def grp(g, c2):
        eidx = lax.iota(i32, L) + g * L
        for hl in range(HEADS // NC):
          hv = jnp.full((L,), hbase + hl, i32)
          ev = plsc.load_gather(exb, [eidx, hv])
          dv = plsc.load_gather(d0b, [eidx, hv]) + \
              plsc.load_gather(d1b, [eidx, hv])
          al = ev / (dv + 1e-16)
          for c in range(PH):
            flv = jnp.full((L,), hl * PH + c, i32)
            v = plsc.load_gather(xlb, [eidx, flv]) * al
            plsc.store_scatter(msgb, [eidx, flv], v)
        return c2
      lax.fori_loop(0, B // L, grp, 0)
      pltpu.sync_copy(msgb, out_sh.at[didx], add=True)
      return carry
    lax.fori_loop(0, nblk, blk, 0)
    plsc.subcore_barrier()
    pltpu.sync_copy(out_sh.at[pl.ds(r0, ROWS_PT)],
                    out_h.at[pl.ds(cid * N_ACC + r0, ROWS_PT)])

  scratch = [pltpu.VMEM((B,), i32), pltpu.VMEM((B,), i32),
             pltpu.VMEM((B,), i32), pltpu.VMEM((B,), i32),
             pltpu.VMEM((B, tw), f32), pltpu.VMEM((B, WST), f32),
             pltpu.VMEM((B, WST), f32), pltpu.VMEM((B, WST), f32),
             pltpu.VMEM((B, tw), f32),
             pltpu.VMEM_SHARED((N_ACC, tw), f32), pltpu.SemaphoreType.DMA]

  return pl.kernel(
      body,
      out_type=jax.ShapeDtypeStruct((2 * N_ACC, tw), f32),
      mesh=plsc.VectorSubcoreMesh(**_MESH),
      scratch_types=scratch,
      compiler_params=pltpu.CompilerParams(needs_layout_passes=False, use_tc_tiling_on_sc=False),
  )


@functools.lru_cache(maxsize=None)
def _make_pass_b1(ep):
  ca = ep // NW
  nblk = ca // B
  tw = OUT_CH

  def body(src_h, dst_h, xl_h, ex_h, den_h, z_h, out_h,
           sidx, didx, didxb, xlb, exb, d0b, d1b, msgb, out_sh, sem):
    cid = lax.axis_index("c")
    sid = lax.axis_index("s")
    wid = sid * NC + cid
    r0 = sid * ROWS_PT
    _zero_rows(z_h, out_sh, r0)
    plsc.subcore_barrier()

    def blk(i, carry):
      base = wid * ca + i * B
      pltpu.sync_copy(src_h.at[pl.ds(base, B)], sidx)
      pltpu.sync_copy(dst_h.at[pl.ds(base, B)], didx)
      for k in range(B // L):
        didxb[pl.ds(k * L, L)] = didx[pl.ds(k * L, L)] + N_ACC
      cps = [pltpu.async_copy(xl_h.at[sidx], xlb, sem),
             pltpu.async_copy(den_h.at[didx], d0b, sem),
             pltpu.async_copy(den_h.at[didxb], d1b, sem)]
      pltpu.sync_copy(ex_h.at[pl.ds(base, B)], exb)
      for cp in cps:
        cp.wait()

      def grp(g, c2):
        eidx = lax.iota(i32, L) + g * L
        hv = jnp.zeros((L,), i32)
        ev = plsc.load_gather(exb, [eidx, hv])
        dv = plsc.load_gather(d0b, [eidx, hv]) + \
            plsc.load_gather(d1b, [eidx, hv])
        al = ev / (dv + 1e-16)
        for c in range(tw):
          flv = jnp.full((L,), c, i32)
          v = plsc.load_gather(xlb, [eidx, flv]) * al
          plsc.store_scatter(msgb, [eidx, flv], v)
        return c2
      lax.fori_loop(0, B // L, grp, 0)
      pltpu.sync_copy(msgb, out_sh.at[didx], add=True)
      return carry
    lax.fori_loop(0, nblk, blk, 0)
    plsc.subcore_barrier()
    pltpu.sync_copy(out_sh.at[pl.ds(r0, ROWS_PT)],
                    out_h.at[pl.ds(cid * N_ACC + r0, ROWS_PT)])

  scratch = [pltpu.VMEM((B,), i32), pltpu.VMEM((B,), i32),
             pltpu.VMEM((B,), i32),
             pltpu.VMEM((B, tw), f32), pltpu.VMEM((B, WST), f32),
             pltpu.VMEM((B, WST), f32), pltpu.VMEM((B, WST), f32),
             pltpu.VMEM((B, tw), f32),
             pltpu.VMEM_SHARED((N_ACC, tw), f32), pltpu.SemaphoreType.DMA]

  return pl.kernel(
      body,
      out_type=jax.ShapeDtypeStruct((2 * N_ACC, tw), f32),
      mesh=plsc.VectorSubcoreMesh(**_MESH),
      scratch_types=scratch,
      compiler_params=pltpu.CompilerParams(needs_layout_passes=False, use_tc_tiling_on_sc=False),
  )


def kernel(x, edge_index0, edge_index1, W_l0, W_r0, att0, b0,
           W_l1, W_r1, att1, b1):
  ep0 = _ceil_to(edge_index0.shape[1] + N, NW * B)
  ep1 = _ceil_to(edge_index1.shape[1] + N, NW * B)
  src0, dst0 = _prep_edges(edge_index0, ep0)
  src1, dst1 = _prep_edges(edge_index1, ep1)

  z16 = jnp.zeros((N_ACC, WST), f32)
  z64 = jnp.zeros((N_ACC, OUT_CH), f32)
  z128 = jnp.zeros((N_ACC, 128), f32)

  xl0, xr0 = _tc_project0(x, W_l0, W_r0)
  ex0, den0 = _make_pass_a(ep0, HEADS, PH, 128, 2)(
      src0, dst0, xl0, xr0, att0.reshape(-1), z16)
  h0 = _make_pass_b0(ep0)(src0, dst0, xl0, ex0, den0, z128)

  xl1, xr1 = _tc_mid(h0, b0, W_l1, W_r1)
  ex1, den1 = _make_pass_a(ep1, 1, OUT_CH, OUT_CH, 1)(
      src1, dst1, xl1, xr1, att1.reshape(-1), z16)
  out1 = _make_pass_b1(ep1)(src1, dst1, xl1, ex1, den1, z64)

  return _tc_final(out1, b1)


# speedup vs baseline: 9.7446x; 1.6833x over previous
import functools

import jax
import jax.numpy as jnp
from jax import lax
from jax.experimental import pallas as pl
from jax.experimental.pallas import tpu as pltpu
from jax.experimental.pallas import tpu_sc as plsc

N = 10000
HID = 256
HEADS = 8
PH = 32
OUT_CH = 64

NC = 2
NS = 16
L = 16
NW = NC * NS
B = 128
N_ACC = 10240
DUMMY = N
WST = 16
f32 = jnp.float32
i32 = jnp.int32

ROWS_PT = N_ACC // NS


def _ceil_to(a, m):
  return ((a + m - 1) // m) * m


def _prep_edges(ei, ep):
  loop = jnp.arange(N, dtype=i32)
  src = jnp.concatenate([ei[0].astype(i32), loop])
  dst = jnp.concatenate([ei[1].astype(i32), loop])
  pad = ep - src.shape[0]
  src = jnp.concatenate([src, jnp.zeros((pad,), i32)])
  dst = jnp.concatenate([dst, jnp.full((pad,), DUMMY, i32)])
  return src, dst


_DN = (((1,), (1,)), ((), ()))


def _proj0_body(x_ref, wl_ref, wr_ref, xl_ref, xr_ref):
  xb = x_ref[...]
  xl_ref[...] = lax.dot_general(xb, wl_ref[...], _DN,
                                preferred_element_type=f32)
  xr_ref[...] = lax.dot_general(xb, wr_ref[...], _DN,
                                preferred_element_type=f32)


def _tc_project0(x, wl, wr):
  grid = (5, 2)
  return pl.pallas_call(
      _proj0_body,
      grid=grid,
      in_specs=[
          pl.BlockSpec((2000, 128), lambda i, j: (i, 0)),
          pl.BlockSpec((128, 128), lambda i, j: (j, 0)),
          pl.BlockSpec((128, 128), lambda i, j: (j, 0)),
      ],
      out_specs=[
          pl.BlockSpec((2000, 128), lambda i, j: (j * 5 + i, 0)),
          pl.BlockSpec((2000, 128), lambda i, j: (j * 5 + i, 0)),
      ],
      out_shape=[
          jax.ShapeDtypeStruct((2 * N, 128), f32),
          jax.ShapeDtypeStruct((2 * N, 128), f32),
      ],
  )(x, wl, wr)


def _mid_body(lo_ref, hi_ref, b0_ref, wl_ref, wr_ref, xl_ref, xr_ref):
  h = jnp.concatenate([lo_ref[...], hi_ref[...]], axis=1) + b0_ref[...]
  h = jnp.where(h > 0, h, jnp.exp(h) - 1.0)
  xl_ref[...] = lax.dot_general(h, wl_ref[...], _DN,
                                preferred_element_type=f32)
  xr_ref[...] = lax.dot_general(h, wr_ref[...], _DN,
                                preferred_element_type=f32)


def _tc_mid(h0, b0, wl1, wr1):
  grid = (5,)
  return pl.pallas_call(
      _mid_body,
      grid=grid,
      in_specs=[
          pl.BlockSpec((2048, 128), lambda i: (i, 0)),
          pl.BlockSpec((2048, 128), lambda i: (i + N_ACC // 2048, 0)),
          pl.BlockSpec((1, HID), lambda i: (0, 0)),
          pl.BlockSpec((OUT_CH, HID), lambda i: (0, 0)),
          pl.BlockSpec((OUT_CH, HID), lambda i: (0, 0)),
      ],
      out_specs=[
          pl.BlockSpec((2048, OUT_CH), lambda i: (i, 0)),
          pl.BlockSpec((2048, OUT_CH), lambda i: (i, 0)),
      ],
      out_shape=[
          jax.ShapeDtypeStruct((N, OUT_CH), f32),
          jax.ShapeDtypeStruct((N, OUT_CH), f32),
      ],
  )(h0, h0, b0.reshape(1, HID), wl1, wr1)


def _final_body(p0_ref, p1_ref, b1_ref, o_ref):
  z = p0_ref[...] + p1_ref[...] + b1_ref[...]
  m = jnp.max(z, axis=1, keepdims=True)
  s = jnp.log(jnp.sum(jnp.exp(z - m), axis=1, keepdims=True))
  o_ref[...] = z - m - s


def _tc_final(out1, b1):
  grid = (5,)
  return pl.pallas_call(
      _final_body,
      grid=grid,
      in_specs=[
          pl.BlockSpec((2048, OUT_CH), lambda i: (i, 0)),
          pl.BlockSpec((2048, OUT_CH), lambda i: (i + N_ACC // 2048, 0)),
          pl.BlockSpec((1, OUT_CH), lambda i: (0, 0)),
      ],
      out_specs=pl.BlockSpec((2048, OUT_CH), lambda i: (i, 0)),
      out_shape=jax.ShapeDtypeStruct((N, OUT_CH), f32),
  )(out1, out1, b1.reshape(1, OUT_CH))


_MESH = dict(core_axis_name="c", subcore_axis_name="s", num_cores=NC,
             num_subcores=NS)


def _lane_take(v, idx):
  dnums = lax.GatherDimensionNumbers(offset_dims=(), collapsed_slice_dims=(0,),
                                     start_index_map=(0,))
  return lax.gather(v, idx[:, None], dnums, (1,),
                    mode=lax.GatherScatterMode.PROMISE_IN_BOUNDS)


def _zero_rows(z_h, sh, r0):
  pltpu.sync_copy(z_h.at[pl.ds(r0, ROWS_PT)], sh.at[pl.ds(r0, ROWS_PT)])


@functools.lru_cache(maxsize=None)
def _make_pass_a(ep, heads, ph, tw, ntab):
  ba = 64
  ca = ep // NW
  nblk = ca // ba
  nck = tw // L

  def body(src_h, dst_h, xl_h, xr_h, att_h, z_h, ex_h, den_h, *scr):
    sidx, didx = scr[0], scr[1]
    xlb = scr[2:2 + ntab]
    xrb = scr[2 + ntab:2 + 2 * ntab]
    xsk = scr[2 + 2 * ntab:2 + 3 * ntab]
    rsk = scr[2 + 3 * ntab:2 + 4 * ntab]
    p = 2 + 4 * ntab
    if ntab == 2:
      sidx2, didx2 = scr[p], scr[p + 1]
      p += 2
    exb, attv, den_sh, sem = scr[p], scr[p + 1], scr[p + 2], scr[p + 3]
    cid = lax.axis_index("c")
    sid = lax.axis_index("s")
    wid = sid * NC + cid
    r0 = sid * ROWS_PT
    pltpu.sync_copy(att_h, attv)
    _zero_rows(z_h, den_sh, r0)
    zero16 = jnp.zeros((L,), f32)
    for r in range(ba):
      exb[r, pl.ds(0, WST)] = zero16
    plsc.subcore_barrier()

    def blk(i, carry):
      base = wid * ca + i * ba
      pltpu.sync_copy(src_h.at[pl.ds(base, ba)], sidx)
      pltpu.sync_copy(dst_h.at[pl.ds(base, ba)], didx)
      if ntab == 2:
        for k in range(ba // L):
          sidx2[pl.ds(k * L, L)] = sidx[pl.ds(k * L, L)] + N
          didx2[pl.ds(k * L, L)] = didx[pl.ds(k * L, L)] + N
      cps = [pltpu.async_copy(xl_h.at[sidx], xlb[0], sem),
             pltpu.async_copy(xr_h.at[didx], xrb[0], sem)]
      if ntab == 2:
        cps.append(pltpu.async_copy(xl_h.at[sidx2], xlb[1], sem))
        cps.append(pltpu.async_copy(xr_h.at[didx2], xrb[1], sem))
      for cp in cps:
        cp.wait()

      def skew(e, c2):
        for t in range(ntab):
          for k in range(nck):
            xsk[t][e, pl.ds(k * L, L)] = xlb[t][e, pl.ds(k * L, L)]
            rsk[t][e, pl.ds(k * L, L)] = xrb[t][e, pl.ds(k * L, L)]
        return c2
      lax.fori_loop(0, ba, skew, 0)

      def grp(g, c2):
        eidx = lax.iota(i32, L) + g * L
        for h in range(heads):
          acc = jnp.zeros((L,), f32)
          for k in range(ph // L):
            av = attv[pl.ds(h * ph + k * L, L)]
            for j in range(L):
              f = h * ph + k * L + j
              t, fl = divmod(f, tw)
              flv = jnp.full((L,), fl, i32)
              s = plsc.load_gather(xsk[t], [eidx, flv]) + \
                  plsc.load_gather(rsk[t], [eidx, flv])
              s = jnp.maximum(s, s * 0.2)
              acc = acc + s * av[j]
          plsc.store_scatter(exb, [eidx, jnp.full((L,), h, i32)],
                             jnp.exp(acc))
        return c2
      lax.fori_loop(0, ba // L, grp, 0)
      pltpu.sync_copy(exb, ex_h.at[pl.ds(base, ba)])
      pltpu.sync_copy(exb, den_sh.at[didx], add=True)
      return carry
    lax.fori_loop(0, nblk, blk, 0)
    plsc.subcore_barrier()
    pltpu.sync_copy(den_sh.at[pl.ds(r0, ROWS_PT)],
                    den_h.at[pl.ds(cid * N_ACC + r0, ROWS_PT)])

  scratch = [pltpu.VMEM((ba,), i32), pltpu.VMEM((ba,), i32)]
  scratch += [pltpu.VMEM((ba, tw), f32) for _ in range(2 * ntab)]
  scratch += [pltpu.VMEM((ba, tw + 1), f32) for _ in range(2 * ntab)]
  if ntab == 2:
    scratch += [pltpu.VMEM((ba,), i32), pltpu.VMEM((ba,), i32)]
  scratch += [pltpu.VMEM((ba, WST), f32), pltpu.VMEM((heads * ph,), f32),
              pltpu.VMEM_SHARED((N_ACC, WST), f32), pltpu.SemaphoreType.DMA]

  return pl.kernel(
      body,
      out_type=[jax.ShapeDtypeStruct((ep, WST), f32),
                jax.ShapeDtypeStruct((2 * N_ACC, WST), f32)],
      mesh=plsc.VectorSubcoreMesh(**_MESH),
      scratch_types=scratch,
      compiler_params=pltpu.CompilerParams(needs_layout_passes=False, use_tc_tiling_on_sc=False),
  )


@functools.lru_cache(maxsize=None)
def _make_pass_b0(ep):
  cb = ep // NS
  nblk = cb // B
  tw = 128

  def body(src_h, dst_h, xl_h, ex_h, den_h, z_h, out_h,
           sidx, didx, gidx, didxb, xlb, exb, d0b, d1b, msgb, out_sh, sem):
    cid = lax.axis_index("c")
    sid = lax.axis_index("s")
    r0 = sid * ROWS_PT
    _zero_rows(z_h, out_sh, r0)
    plsc.subcore_barrier()
    hbase = cid * (HEADS // NC)

    def blk(i, carry):
      base = sid * cb + i * B
      pltpu.sync_copy(src_h.at[pl.ds(base, B)], sidx)
      pltpu.sync_copy(dst_h.at[pl.ds(base, B)], didx)
      coff = cid * N
      for k in range(B // L):
        gidx[pl.ds(k * L, L)] = sidx[pl.ds(k * L, L)] + coff
        didxb[pl.ds(k * L, L)] = didx[pl.ds(k * L, L)] + N_ACC
      cps = [pltpu.async_copy(xl_h.at[gidx], xlb, sem),
             pltpu.async_copy(den_h.at[didx], d0b, sem),
             pltpu.async_copy(den_h.at[didxb], d1b, sem)]
      pltpu.sync_copy(ex_h.at[pl.ds(base, B)], exb)
      for cp in cps:
        cp.wait()

      def edge(e, c2):
        exr = exb[e, pl.ds(0, WST)]
        dr = d0b[e, pl.ds(0, WST)] + d1b[e, pl.ds(0, WST)]
        al = exr / (dr + 1e-16)
        for hl in range(HEADS // NC):
          hv = jnp.full((L,), hbase + hl, i32)
          bc = _lane_take(al, hv)
          for k in range(PH // L):
            c0 = hl * PH + k * L
            msgb[e, pl.ds(c0, L)] = xlb[e, pl.ds(c0, L)] * bc
        return c2
      lax.fori_loop(0, B, edge, 0)
      pltpu.sync_copy(msgb, out_sh.at[didx], add=True)
      return carry
    lax.fori_loop(0, nblk, blk, 0)
    plsc.subcore_barrier()
    pltpu.sync_copy(out_sh.at[pl.ds(r0, ROWS_PT)],
                    out_h.at[pl.ds(cid * N_ACC + r0, ROWS_PT)])

  scratch = [pltpu.VMEM((B,), i32), pltpu.VMEM((B,), i32),
             pltpu.VMEM((B,), i32), pltpu.VMEM((B,), i32),
             pltpu.VMEM((B, tw), f32), pltpu.VMEM((B, WST), f32),
             pltpu.VMEM((B, WST), f32), pltpu.VMEM((B, WST), f32),
             pltpu.VMEM((B, tw), f32),
             pltpu.VMEM_SHARED((N_ACC, tw), f32), pltpu.SemaphoreType.DMA]

  return pl.kernel(
      body,
      out_type=jax.ShapeDtypeStruct((2 * N_ACC, tw), f32),
      mesh=plsc.VectorSubcoreMesh(**_MESH),
      scratch_types=scratch,
      compiler_params=pltpu.CompilerParams(needs_layout_passes=False, use_tc_tiling_on_sc=False),
  )


@functools.lru_cache(maxsize=None)
def _make_pass_b1(ep):
  ca = ep // NW
  nblk = ca // B
  tw = OUT_CH

  def body(src_h, dst_h, xl_h, ex_h, den_h, z_h, out_h,
           sidx, didx, didxb, xlb, exb, d0b, d1b, msgb, out_sh, sem):
    cid = lax.axis_index("c")
    sid = lax.axis_index("s")
    wid = sid * NC + cid
    r0 = sid * ROWS_PT
    _zero_rows(z_h, out_sh, r0)
    plsc.subcore_barrier()

    def blk(i, carry):
      base = wid * ca + i * B
      pltpu.sync_copy(src_h.at[pl.ds(base, B)], sidx)
      pltpu.sync_copy(dst_h.at[pl.ds(base, B)], didx)
      for k in range(B // L):
        didxb[pl.ds(k * L, L)] = didx[pl.ds(k * L, L)] + N_ACC
      cps = [pltpu.async_copy(xl_h.at[sidx], xlb, sem),
             pltpu.async_copy(den_h.at[didx], d0b, sem),
             pltpu.async_copy(den_h.at[didxb], d1b, sem)]
      pltpu.sync_copy(ex_h.at[pl.ds(base, B)], exb)
      for cp in cps:
        cp.wait()

      def edge(e, c2):
        exr = exb[e, pl.ds(0, WST)]
        dr = d0b[e, pl.ds(0, WST)] + d1b[e, pl.ds(0, WST)]
        al = exr / (dr + 1e-16)
        bc = _lane_take(al, jnp.zeros((L,), i32))
        for k in range(tw // L):
          msgb[e, pl.ds(k * L, L)] = xlb[e, pl.ds(k * L, L)] * bc
        return c2
      lax.fori_loop(0, B, edge, 0)
      pltpu.sync_copy(msgb, out_sh.at[didx], add=True)
      return carry
    lax.fori_loop(0, nblk, blk, 0)
    plsc.subcore_barrier()
    pltpu.sync_copy(out_sh.at[pl.ds(r0, ROWS_PT)],
                    out_h.at[pl.ds(cid * N_ACC + r0, ROWS_PT)])

  scratch = [pltpu.VMEM((B,), i32), pltpu.VMEM((B,), i32),
             pltpu.VMEM((B,), i32),
             pltpu.VMEM((B, tw), f32), pltpu.VMEM((B, WST), f32),
             pltpu.VMEM((B, WST), f32), pltpu.VMEM((B, WST), f32),
             pltpu.VMEM((B, tw), f32),
             pltpu.VMEM_SHARED((N_ACC, tw), f32), pltpu.SemaphoreType.DMA]

  return pl.kernel(
      body,
      out_type=jax.ShapeDtypeStruct((2 * N_ACC, tw), f32),
      mesh=plsc.VectorSubcoreMesh(**_MESH),
      scratch_types=scratch,
      compiler_params=pltpu.CompilerParams(needs_layout_passes=False, use_tc_tiling_on_sc=False),
  )


def kernel(x, edge_index0, edge_index1, W_l0, W_r0, att0, b0,
           W_l1, W_r1, att1, b1):
  ep0 = _ceil_to(edge_index0.shape[1] + N, NW * B)
  ep1 = _ceil_to(edge_index1.shape[1] + N, NW * B)
  src0, dst0 = _prep_edges(edge_index0, ep0)
  src1, dst1 = _prep_edges(edge_index1, ep1)

  z16 = jnp.zeros((N_ACC, WST), f32)
  z64 = jnp.zeros((N_ACC, OUT_CH), f32)
  z128 = jnp.zeros((N_ACC, 128), f32)

  xl0, xr0 = _tc_project0(x, W_l0, W_r0)
  ex0, den0 = _make_pass_a(ep0, HEADS, PH, 128, 2)(
      src0, dst0, xl0, xr0, att0.reshape(-1), z16)
  h0 = _make_pass_b0(ep0)(src0, dst0, xl0, ex0, den0, z128)

  xl1, xr1 = _tc_mid(h0, b0, W_l1, W_r1)
  ex1, den1 = _make_pass_a(ep1, 1, OUT_CH, OUT_CH, 1)(
      src1, dst1, xl1, xr1, att1.reshape(-1), z16)
  out1 = _make_pass_b1(ep1)(src1, dst1, xl1, ex1, den1, z64)

  return _tc_final(out1, b1)
